# baseline (speedup 1.0000x reference)
import functools

import jax
import jax.numpy as jnp
from jax import lax
from jax.experimental import pallas as pl
from jax.experimental.pallas import tpu as pltpu
from jax.experimental.pallas import tpu_sc as plsc

B = 16384
D = 64
V = 100000
NC, NS = 2, 16
NW = NC * NS
BPW = B // NW
CHUNK = 128
NCH = BPW // CHUNK
H1 = 128
H2 = 64
PACK_BK = 2000


def _pack_body(ut, ub, it, ib, uat, uab, iat, iab, uo_ref, io_ref, uao_ref,
               ia_o_ref):
    for top, bot, dst in ((ut, ub, uo_ref), (it, ib, io_ref),
                          (uat, uab, uao_ref), (iat, iab, ia_o_ref)):
        dst[...] = jnp.concatenate([top[...], bot[...]], axis=1)


def _tc_pack(um, im, ua, ia):
    nblk = (V // 2) // PACK_BK
    spec_top = pl.BlockSpec((PACK_BK, D), lambda i: (i, 0))
    spec_bot = pl.BlockSpec((PACK_BK, D), lambda i: (i + nblk, 0))
    spec_out = pl.BlockSpec((PACK_BK, 128), lambda i: (i, 0))
    shp = jax.ShapeDtypeStruct((V // 2, 128), jnp.float32)
    return pl.pallas_call(
        _pack_body,
        grid=(nblk,),
        in_specs=[spec_top, spec_bot] * 4,
        out_specs=[spec_out] * 4,
        out_shape=[shp] * 4,
    )(um, um, im, im, ua, ua, ia, ia)


def _sc_gather(idx3, um, im, ua, ia):
    mesh = plsc.VectorSubcoreMesh(core_axis_name="c", subcore_axis_name="s")

    @functools.partial(
        pl.kernel,
        out_type=jax.ShapeDtypeStruct((6, B, 128), jnp.float32),
        mesh=mesh,
        scratch_types=[
            pltpu.VMEM((6, NCH, CHUNK), jnp.int32),
            pltpu.VMEM((2, CHUNK, 128), jnp.float32),
            pltpu.SemaphoreType.DMA,
            pltpu.SemaphoreType.DMA,
            pltpu.SemaphoreType.DMA,
        ],
        compiler_params=pltpu.CompilerParams(use_tc_tiling_on_sc=False),
    )
    def k(idx_hbm, um_hbm, im_hbm, ua_hbm, ia_hbm, out_hbm, idx_v, buf, gsem,
          ssem0, ssem1):
        wid = lax.axis_index("s") * NC + lax.axis_index("c")
        gbase = wid * NCH
        pltpu.sync_copy(idx_hbm.at[pl.ds(0, 6), pl.ds(gbase, NCH)], idx_v)

        tables = [um_hbm, im_hbm, ua_hbm, ua_hbm, ia_hbm, ia_hbm]
        steps = [(tables[s], s, c) for s in range(6) for c in range(NCH)]
        ssems = [ssem0, ssem1]
        store = [None, None]
        gath = [None, None]
        for n in range(len(steps) + 1):
            if n < len(steps):
                bsel = n & 1
                if store[bsel] is not None:
                    store[bsel].wait()
                tbl, s, c = steps[n]
                gath[bsel] = pltpu.async_copy(
                    tbl.at[idx_v.at[s, c]], buf.at[bsel], gsem)
            if n >= 1:
                pb = (n - 1) & 1
                gath[pb].wait()
                _, s, c = steps[n - 1]
                store[pb] = pltpu.async_copy(
                    buf.at[pb],
                    out_hbm.at[s, pl.ds(wid * BPW + c * CHUNK, CHUNK)],
                    ssems[pb])
        for bsel in range(2):
            if store[bsel] is not None:
                store[bsel].wait()

    return k(idx3, um, im, ua, ia)


def _tc_body(wos_ref, bo_ref, g_ref, par_ref, w1_ref, b1_ref, w2_ref, b2_ref,
             woh_ref, out_ref):
    def sel(s):
        gp = g_ref[s]
        odd = par_ref[:, s:s + 1] > 0
        return jnp.where(odd, gp[:, D:], gp[:, :D])

    u = sel(0)
    it = sel(1)
    h = jnp.dot(u, w1_ref[:D, :], preferred_element_type=jnp.float32)
    h = h + jnp.dot(it, w1_ref[D:, :], preferred_element_type=jnp.float32)
    h = jnp.maximum(h + b1_ref[...], 0.0)
    h = jnp.dot(h, w2_ref[...], preferred_element_type=jnp.float32)
    h = jnp.maximum(h + b2_ref[...], 0.0)

    def cos(x, y):
        num = jnp.sum(x * y, axis=1)
        na = jnp.sqrt(jnp.sum(x * x, axis=1))
        nb = jnp.sqrt(jnp.sum(y * y, axis=1))
        return num / jnp.maximum(na * nb, 1e-8)

    s0 = cos(sel(2), sel(4))
    s1 = cos(sel(3), sel(5))
    logit = jnp.sum(h * woh_ref[...], axis=1)
    out_ref[...] = logit + s0 * wos_ref[0] + s1 * wos_ref[1] + bo_ref[0]


def _tc_mlp(g, par, w1, b1, w2, b2, wos, woh, bo, interpret=False):
    bk = 2048
    return pl.pallas_call(
        _tc_body,
        grid=(B // bk,),
        in_specs=[
            pl.BlockSpec(memory_space=pltpu.SMEM),
            pl.BlockSpec(memory_space=pltpu.SMEM),
            pl.BlockSpec((6, bk, 128), lambda i: (0, i, 0)),
            pl.BlockSpec((bk, 6), lambda i: (i, 0)),
            pl.BlockSpec((H1, H1), lambda i: (0, 0)),
            pl.BlockSpec((1, H1), lambda i: (0, 0)),
            pl.BlockSpec((H1, H2), lambda i: (0, 0)),
            pl.BlockSpec((1, H2), lambda i: (0, 0)),
            pl.BlockSpec((1, H2), lambda i: (0, 0)),
        ],
        out_specs=pl.BlockSpec((bk,), lambda i: (i,)),
        out_shape=jax.ShapeDtypeStruct((B,), jnp.float32),
        interpret=interpret,
    )(wos, bo, g, par, w1, b1, w2, b2, woh)


def kernel(user, item, emb_user_mlp, emb_item_mlp, emb_user_ac, emb_item_ac,
           W1, b1, W2, b2, Wo, bo):
    idx = jnp.stack([
        user[:, 0], item[:, 0],
        user[:, 1], user[:, 2],
        item[:, 1], item[:, 2],
    ])
    idx3 = (idx % (V // 2)).reshape(6, B // CHUNK, CHUNK)
    par = (idx // (V // 2)).T
    pum, pim, pua, pia = _tc_pack(emb_user_mlp, emb_item_mlp, emb_user_ac,
                                  emb_item_ac)
    g = _sc_gather(idx3, pum, pim, pua, pia)
    logit = _tc_mlp(g, par, W1, b1.reshape(1, H1), W2, b2.reshape(1, H2),
                    Wo[:2, 0], Wo[2:, 0].reshape(1, H2), bo)
    return logit.reshape(B, 1)


# speedup vs baseline: 1.1822x; 1.1822x over previous
import functools

import jax
import jax.numpy as jnp
from jax import lax
from jax.experimental import pallas as pl
from jax.experimental.pallas import tpu as pltpu
from jax.experimental.pallas import tpu_sc as plsc

B = 16384
D = 64
V = 100000
NC, NS = 2, 16
NW = NC * NS
BPW = B // NW
CHUNK = 128
NCH = BPW // CHUNK
H1 = 128
H2 = 64
PACK_BK = 4000


def _pack_body(um_ref, ua_ref, im_ref, ia_ref, uc_ref, ic_ref):
    uc_ref[...] = jnp.concatenate([um_ref[...], ua_ref[...]], axis=1)
    ic_ref[...] = jnp.concatenate([im_ref[...], ia_ref[...]], axis=1)


def _tc_pack(um, ua, im, ia):
    spec_in = pl.BlockSpec((PACK_BK, D), lambda i: (i, 0))
    spec_out = pl.BlockSpec((PACK_BK, 128), lambda i: (i, 0))
    shp = jax.ShapeDtypeStruct((V, 128), jnp.float32)
    return pl.pallas_call(
        _pack_body,
        grid=(V // PACK_BK,),
        in_specs=[spec_in] * 4,
        out_specs=[spec_out] * 2,
        out_shape=[shp] * 2,
    )(um, ua, im, ia)


def _sc_gather(idx3, uc, ic):
    mesh = plsc.VectorSubcoreMesh(core_axis_name="c", subcore_axis_name="s")

    @functools.partial(
        pl.kernel,
        out_type=(jax.ShapeDtypeStruct((B, 128), jnp.float32),
                  jax.ShapeDtypeStruct((2, B, 128), jnp.float32)),
        mesh=mesh,
        scratch_types=[
            pltpu.VMEM((6, NCH, CHUNK), jnp.int32),
            pltpu.VMEM((2, CHUNK, 128), jnp.float32),
            pltpu.SemaphoreType.DMA,
            pltpu.SemaphoreType.DMA,
            pltpu.SemaphoreType.DMA,
        ],
        compiler_params=pltpu.CompilerParams(use_tc_tiling_on_sc=False),
    )
    def k(idx_hbm, uc_hbm, ic_hbm, gm_hbm, ga_hbm, idx_v, buf, gsem,
          ssem0, ssem1):
        wid = lax.axis_index("s") * NC + lax.axis_index("c")
        gbase = wid * NCH
        pltpu.sync_copy(idx_hbm.at[pl.ds(0, 6), pl.ds(gbase, NCH)], idx_v)

        def dst(s, c):
            rows = pl.ds(wid * BPW + c * CHUNK, CHUNK)
            if s == 0:
                return gm_hbm.at[rows, pl.ds(0, D)]
            if s == 1:
                return gm_hbm.at[rows, pl.ds(D, D)]
            j = (s - 2) & 1
            half = 0 if s < 4 else D
            return ga_hbm.at[j, rows, pl.ds(half, D)]

        tables = [uc_hbm, ic_hbm, uc_hbm, uc_hbm, ic_hbm, ic_hbm]
        halves = [0, 0, D, D, D, D]
        steps = [(s, c) for s in range(6) for c in range(NCH)]
        ssems = [ssem0, ssem1]
        store = [None, None]
        gath = [None, None]
        for n in range(len(steps) + 1):
            if n < len(steps):
                bsel = n & 1
                if store[bsel] is not None:
                    store[bsel].wait()
                s, c = steps[n]
                gath[bsel] = pltpu.async_copy(
                    tables[s].at[idx_v.at[s, c]], buf.at[bsel], gsem)
            if n >= 1:
                pb = (n - 1) & 1
                gath[pb].wait()
                s, c = steps[n - 1]
                store[pb] = pltpu.async_copy(
                    buf.at[pb, :, pl.ds(halves[s], D)], dst(s, c), ssems[pb])
        for bsel in range(2):
            if store[bsel] is not None:
                store[bsel].wait()

    return k(idx3, uc, ic)


def _tc_body(wos_ref, bo_ref, gm_ref, ga_ref, w1_ref, b1_ref, w2_ref, b2_ref,
             woh_ref, out_ref):
    h = jnp.dot(gm_ref[...], w1_ref[...], preferred_element_type=jnp.float32)
    h = jnp.maximum(h + b1_ref[...], 0.0)
    h = jnp.dot(h, w2_ref[...], preferred_element_type=jnp.float32)
    h = jnp.maximum(h + b2_ref[...], 0.0)

    def cos(j):
        x = ga_ref[j, :, :D]
        y = ga_ref[j, :, D:]
        num = jnp.sum(x * y, axis=1)
        na = jnp.sqrt(jnp.sum(x * x, axis=1))
        nb = jnp.sqrt(jnp.sum(y * y, axis=1))
        return num / jnp.maximum(na * nb, 1e-8)

    logit = jnp.sum(h * woh_ref[...], axis=1)
    out_ref[...] = (logit + cos(0) * wos_ref[0] + cos(1) * wos_ref[1]
                    + bo_ref[0])


def _tc_mlp(gm, ga, w1, b1, w2, b2, wos, woh, bo, interpret=False):
    bk = 2048
    return pl.pallas_call(
        _tc_body,
        grid=(B // bk,),
        in_specs=[
            pl.BlockSpec(memory_space=pltpu.SMEM),
            pl.BlockSpec(memory_space=pltpu.SMEM),
            pl.BlockSpec((bk, 128), lambda i: (i, 0)),
            pl.BlockSpec((2, bk, 128), lambda i: (0, i, 0)),
            pl.BlockSpec((H1, H1), lambda i: (0, 0)),
            pl.BlockSpec((1, H1), lambda i: (0, 0)),
            pl.BlockSpec((H1, H2), lambda i: (0, 0)),
            pl.BlockSpec((1, H2), lambda i: (0, 0)),
            pl.BlockSpec((1, H2), lambda i: (0, 0)),
        ],
        out_specs=pl.BlockSpec((bk,), lambda i: (i,)),
        out_shape=jax.ShapeDtypeStruct((B,), jnp.float32),
        interpret=interpret,
    )(wos, bo, gm, ga, w1, b1, w2, b2, woh)


def kernel(user, item, emb_user_mlp, emb_item_mlp, emb_user_ac, emb_item_ac,
           W1, b1, W2, b2, Wo, bo):
    idx3 = jnp.stack([
        user[:, 0], item[:, 0],
        user[:, 1], user[:, 2],
        item[:, 1], item[:, 2],
    ]).reshape(6, B // CHUNK, CHUNK)
    uc, ic = _tc_pack(emb_user_mlp, emb_user_ac, emb_item_mlp, emb_item_ac)
    gm, ga = _sc_gather(idx3, uc, ic)
    logit = _tc_mlp(gm, ga, W1, b1.reshape(1, H1), W2, b2.reshape(1, H2),
                    Wo[:2, 0], Wo[2:, 0].reshape(1, H2), bo)
    return logit.reshape(B, 1)


# speedup vs baseline: 2.2555x; 1.9079x over previous
import functools

import jax
import jax.numpy as jnp
from jax import lax
from jax.experimental import pallas as pl
from jax.experimental.pallas import tpu as pltpu
from jax.experimental.pallas import tpu_sc as plsc

B = 16384
D = 64
V = 100000
NC, NS = 2, 16
NW = NC * NS
BPW = B // NW
CHUNK = 128
NCH = BPW // CHUNK
H1 = 128
H2 = 64
PACK_BK = 4096


def _pack_body(um_ref, ua_ref, im_ref, ia_ref, uc_ref, ic_ref):
    eye = jnp.eye(D, dtype=jnp.float32)

    def tr(x):
        return jax.lax.dot_general(x, eye, (((0,), (0,)), ((), ())),
                                   preferred_element_type=jnp.float32)

    uc_ref[...] = jnp.concatenate(
        [tr(um_ref[...]), tr(ua_ref[...])], axis=1)
    ic_ref[...] = jnp.concatenate(
        [tr(im_ref[...]), tr(ia_ref[...])], axis=1)


def _tc_pack(um_t, ua_t, im_t, ia_t):
    spec_in = pl.BlockSpec((D, PACK_BK), lambda i: (0, i))
    spec_out = pl.BlockSpec((PACK_BK, 128), lambda i: (i, 0))
    shp = jax.ShapeDtypeStruct((V, 128), jnp.float32)
    return pl.pallas_call(
        _pack_body,
        grid=(pl.cdiv(V, PACK_BK),),
        in_specs=[spec_in] * 4,
        out_specs=[spec_out] * 2,
        out_shape=[shp] * 2,
    )(um_t, ua_t, im_t, ia_t)


def _sc_gather(idx3, uc, ic):
    mesh = plsc.VectorSubcoreMesh(core_axis_name="c", subcore_axis_name="s")

    @functools.partial(
        pl.kernel,
        out_type=(jax.ShapeDtypeStruct((B, 128), jnp.float32),
                  jax.ShapeDtypeStruct((2, B, 128), jnp.float32)),
        mesh=mesh,
        scratch_types=[
            pltpu.VMEM((6, NCH, CHUNK), jnp.int32),
            pltpu.VMEM((2, CHUNK, 128), jnp.float32),
            pltpu.SemaphoreType.DMA,
            pltpu.SemaphoreType.DMA,
            pltpu.SemaphoreType.DMA,
        ],
        compiler_params=pltpu.CompilerParams(use_tc_tiling_on_sc=False),
    )
    def k(idx_hbm, uc_hbm, ic_hbm, gm_hbm, ga_hbm, idx_v, buf, gsem,
          ssem0, ssem1):
        wid = lax.axis_index("s") * NC + lax.axis_index("c")
        gbase = wid * NCH
        pltpu.sync_copy(idx_hbm.at[pl.ds(0, 6), pl.ds(gbase, NCH)], idx_v)

        def dst(s, c):
            rows = pl.ds(wid * BPW + c * CHUNK, CHUNK)
            if s == 0:
                return gm_hbm.at[rows, pl.ds(0, D)]
            if s == 1:
                return gm_hbm.at[rows, pl.ds(D, D)]
            j = (s - 2) & 1
            half = 0 if s < 4 else D
            return ga_hbm.at[j, rows, pl.ds(half, D)]

        tables = [uc_hbm, ic_hbm, uc_hbm, uc_hbm, ic_hbm, ic_hbm]
        halves = [0, 0, D, D, D, D]
        steps = [(s, c) for s in range(6) for c in range(NCH)]
        ssems = [ssem0, ssem1]
        store = [None, None]
        gath = [None, None]
        for n in range(len(steps) + 1):
            if n < len(steps):
                bsel = n & 1
                if store[bsel] is not None:
                    store[bsel].wait()
                s, c = steps[n]
                gath[bsel] = pltpu.async_copy(
                    tables[s].at[idx_v.at[s, c]], buf.at[bsel], gsem)
            if n >= 1:
                pb = (n - 1) & 1
                gath[pb].wait()
                s, c = steps[n - 1]
                store[pb] = pltpu.async_copy(
                    buf.at[pb, :, pl.ds(halves[s], D)], dst(s, c), ssems[pb])
        for bsel in range(2):
            if store[bsel] is not None:
                store[bsel].wait()

    return k(idx3, uc, ic)


def _tc_body(wos_ref, bo_ref, gm_ref, ga_ref, w1_ref, b1_ref, w2_ref, b2_ref,
             woh_ref, out_ref):
    h = jnp.dot(gm_ref[...], w1_ref[...], preferred_element_type=jnp.float32)
    h = jnp.maximum(h + b1_ref[...], 0.0)
    h = jnp.dot(h, w2_ref[...], preferred_element_type=jnp.float32)
    h = jnp.maximum(h + b2_ref[...], 0.0)

    rows = jax.lax.broadcasted_iota(jnp.int32, (3 * D, 3), 0) // D
    cols = jax.lax.broadcasted_iota(jnp.int32, (3 * D, 3), 1)
    seg = jnp.where(rows == cols, 1.0, 0.0).astype(jnp.float32)

    def cos(j):
        x = ga_ref[j, :, :D]
        y = ga_ref[j, :, D:]
        p = jnp.concatenate([x * y, x * x, y * y], axis=1)
        m = jnp.dot(p, seg, preferred_element_type=jnp.float32)
        num = m[:, 0:1]
        den = jnp.sqrt(m[:, 1:2]) * jnp.sqrt(m[:, 2:3])
        return num / jnp.maximum(den, 1e-8)

    logit = jnp.dot(h, woh_ref[...], preferred_element_type=jnp.float32)
    out_ref[...] = (logit + cos(0) * wos_ref[0] + cos(1) * wos_ref[1]
                    + bo_ref[0])


def _tc_mlp(gm, ga, w1, b1, w2, b2, wos, woh, bo, interpret=False):
    bk = 2048
    return pl.pallas_call(
        _tc_body,
        grid=(B // bk,),
        in_specs=[
            pl.BlockSpec(memory_space=pltpu.SMEM),
            pl.BlockSpec(memory_space=pltpu.SMEM),
            pl.BlockSpec((bk, 128), lambda i: (i, 0)),
            pl.BlockSpec((2, bk, 128), lambda i: (0, i, 0)),
            pl.BlockSpec((H1, H1), lambda i: (0, 0)),
            pl.BlockSpec((1, H1), lambda i: (0, 0)),
            pl.BlockSpec((H1, H2), lambda i: (0, 0)),
            pl.BlockSpec((1, H2), lambda i: (0, 0)),
            pl.BlockSpec((H2, 1), lambda i: (0, 0)),
        ],
        out_specs=pl.BlockSpec((bk, 1), lambda i: (i, 0)),
        out_shape=jax.ShapeDtypeStruct((B, 1), jnp.float32),
        interpret=interpret,
    )(wos, bo, gm, ga, w1, b1, w2, b2, woh)


def kernel(user, item, emb_user_mlp, emb_item_mlp, emb_user_ac, emb_item_ac,
           W1, b1, W2, b2, Wo, bo):
    idx3 = jnp.stack([
        user[:, 0], item[:, 0],
        user[:, 1], user[:, 2],
        item[:, 1], item[:, 2],
    ]).reshape(6, B // CHUNK, CHUNK)
    uc, ic = _tc_pack(emb_user_mlp.T, emb_user_ac.T, emb_item_mlp.T,
                      emb_item_ac.T)
    gm, ga = _sc_gather(idx3, uc, ic)
    return _tc_mlp(gm, ga, W1, b1.reshape(1, H1), W2, b2.reshape(1, H2),
                   Wo[:2, 0], Wo[2:, :], bo)
